# baseline (speedup 1.0000x reference)
import functools

import jax
import jax.numpy as jnp
from jax import lax
from jax.experimental import pallas as pl
from jax.experimental.pallas import tpu as pltpu
from jax.experimental.pallas import tpu_sc as plsc

DELTA_V = 0.5
DELTA_D = 1.5
ALPHA = 1.0
BETA = 1.0
GAMMA = 0.001
NCLS = 16
EPS = 1e-12

FDIM = 32
HDIM = 512
WDIM = 512
NTILE = 16
ROWS_PER_TILE = HDIM // NTILE
CROWS = 2
NCHUNK = ROWS_PER_TILE // CROWS
NGRP_ROW = WDIM // 16

RBLK = 16
SDIM = FDIM + 1

VAR_OFF = 0
SH_WORDS = 2 * NTILE * 16


def _stats_body(emb_ref, lab_ref, out_ref):
    rb = pl.program_id(1)
    cls_col = lax.broadcasted_iota(jnp.int32, (WDIM, NCLS), 1)
    acc = jnp.zeros((FDIM, NCLS), jnp.float32)
    cnt = jnp.zeros((1, NCLS), jnp.float32)
    for r in range(RBLK):
        lab_r = lab_ref[0, r, :]
        onehot = (lab_r[:, None] == cls_col).astype(jnp.float32)
        emb_r = emb_ref[0, :, r, :]
        acc = acc + lax.dot_general(
            emb_r, onehot, (((1,), (0,)), ((), ())),
            preferred_element_type=jnp.float32)
        cnt = cnt + jnp.sum(onehot, axis=0, keepdims=True)

    blk = jnp.concatenate([acc, cnt], axis=0)

    @pl.when(rb == 0)
    def _():
        out_ref[0] = jnp.zeros_like(out_ref[0])

    out_ref[0] += blk


def _stats_call(embeds, labels):
    grid = (4, HDIM // RBLK)
    return pl.pallas_call(
        _stats_body,
        grid=grid,
        in_specs=[
            pl.BlockSpec((1, FDIM, RBLK, WDIM), lambda b, r: (b, 0, r, 0)),
            pl.BlockSpec((1, RBLK, WDIM), lambda b, r: (b, r, 0)),
        ],
        out_specs=pl.BlockSpec((1, SDIM, NCLS), lambda b, r: (b, 0, 0)),
        out_shape=jax.ShapeDtypeStruct((4, SDIM, NCLS), jnp.float32),
        compiler_params=pltpu.CompilerParams(
            dimension_semantics=("arbitrary", "arbitrary")),
    )(embeds, labels)


def _take16(vec, idx):
    dn = lax.GatherDimensionNumbers(offset_dims=(), collapsed_slice_dims=(0,),
                                    start_index_map=(0,))
    return lax.gather(vec, idx[:, None], dn, (1,),
                      mode=lax.GatherScatterMode.PROMISE_IN_BOUNDS)


def _sqrt16(x):
    i = plsc.bitcast(x, jnp.int32)
    i = jnp.int32(0x5F3759DF) - (i >> 1)
    r = plsc.bitcast(i, jnp.float32)
    for _ in range(3):
        r = r * (1.5 - 0.5 * x * r * r)
    return x * r


def _dl_body(emb_hbm, lab_hbm, stats_hbm, out_hbm, sh, stat_v, emb_buf,
             lab_buf, varbuf, vread, obuf, sem_e0, sem_e1, sem_l0, sem_l1):
    c = lax.axis_index("c")
    s = lax.axis_index("s")
    sem_e = (sem_e0, sem_e1)
    sem_l = (sem_l0, sem_l1)
    iota = lax.iota(jnp.int32, 16)
    zeros = jnp.zeros((16,), jnp.float32)
    zi = jnp.zeros((16,), jnp.int32)

    def start_at(bb, base, slot):
        pltpu.async_copy(emb_hbm.at[bb, :, pl.ds(base, CROWS), :],
                         emb_buf.at[slot], sem_e[slot])
        pltpu.async_copy(lab_hbm.at[bb, pl.ds(base, CROWS), :],
                         lab_buf.at[slot], sem_l[slot])

    def wait_slot(slot):
        pltpu.make_async_copy(emb_hbm.at[0, :, pl.ds(0, CROWS), :],
                              emb_buf.at[slot], sem_e[slot]).wait()
        pltpu.make_async_copy(lab_hbm.at[0, pl.ds(0, CROWS), :],
                              lab_buf.at[slot], sem_l[slot]).wait()

    def stream_chunks(b, row0, process, carry_init, primed, tail_b,
                      tail_row0):
        if not primed:
            start_at(b, row0, 0)

        def body(kk, car):
            k0 = kk * 2
            start_at(b, row0 + (k0 + 1) * CROWS, 1)
            wait_slot(0)
            car = process(0, car)
            nk = k0 + 2
            last = nk >= NCHUNK
            bb = jnp.where(last, tail_b, b)
            base = jnp.where(last, tail_row0, row0 + nk * CROWS)
            start_at(bb, base, 0)
            wait_slot(1)
            car = process(1, car)
            return car

        return lax.fori_loop(0, NCHUNK // 2, body, carry_init)

    per_image = []
    row0 = s * ROWS_PER_TILE
    start_at(c * 2, row0, 0)
    for b_local in range(2):
        b = c * 2 + b_local

        pltpu.sync_copy(stats_hbm.at[b], stat_v)

        counts = stat_v[FDIM, :]
        present = counts > 0.0
        pres_f = jnp.where(present, 1.0, 0.0).astype(jnp.float32)
        safe = jnp.where(present, counts, 1.0)
        inv_vec = 1.0 / safe
        C_b = jnp.full((16,), jnp.sum(pres_f), jnp.float32)
        mrows = [stat_v[f, :] / safe for f in range(FDIM)]

        nrm2 = zeros
        for f in range(FDIM):
            nrm2 = nrm2 + mrows[f] * mrows[f]
        norms = _sqrt16(nrm2 + EPS)
        reg_sum = jnp.full((16,), jnp.sum(jnp.where(present, norms, 0.0)),
                           jnp.float32)
        reg_b = jnp.where(C_b > 1.0, reg_sum, 0.0)

        def disrow(i, acc):
            d2 = zeros
            for f in range(FDIM):
                mi = _take16(mrows[f], zi + i)
                d = mrows[f] - mi
                d2 = d2 + d * d
            dmat = _sqrt16(d2 + EPS)
            h = jnp.maximum(DELTA_D - dmat, 0.0)
            h = h * h
            pi = _take16(pres_f, zi + i)
            msk = jnp.where(iota > i, pres_f, 0.0) * pi
            return acc + h * msk

        pair_vec = lax.fori_loop(0, NCLS, disrow, zeros)
        pair_sum = jnp.full((16,), jnp.sum(pair_vec), jnp.float32)
        denom = jnp.maximum(C_b * (C_b - 1.0), 1.0)
        dis_b = jnp.where(C_b > 2.0, pair_sum / denom, 0.0)
        per_image.append((dis_b, reg_b, C_b))

        def proc2(slot, vacc):
            def grp(g, va):
                r = g >> 5
                cc = (g & (NGRP_ROW - 1)) * 16
                lab16 = lab_buf[slot, r, pl.ds(cc, 16)]
                accs = [jnp.full((16,), EPS, jnp.float32),
                        zeros, zeros, zeros]
                for f in range(FDIM):
                    v = emb_buf[slot, f, r, pl.ds(cc, 16)]
                    m = _take16(mrows[f], lab16)
                    d = v - m
                    accs[f % 4] = accs[f % 4] + d * d
                acc = (accs[0] + accs[1]) + (accs[2] + accs[3])
                dist = _sqrt16(acc)
                h = jnp.maximum(dist - DELTA_V, 0.0)
                ic = _take16(inv_vec, lab16)
                return va + h * h * ic

            return plsc.parallel_loop(0, CROWS * NGRP_ROW, unroll=2,
                                      carry=vacc)(grp)

        var_vec = stream_chunks(b, row0, proc2, zeros, primed=True,
                                tail_b=jnp.minimum(b + 1, 2 * c + 1),
                                tail_row0=row0)
        varbuf[...] = var_vec
        slot = VAR_OFF + (b_local * NTILE + s) * 16
        pltpu.sync_copy(varbuf, sh.at[pl.ds(slot, 16)])

    wait_slot(0)
    plsc.subcore_barrier()

    pltpu.sync_copy(sh.at[pl.ds(VAR_OFF, 2 * NTILE * 16)], vread)
    loss = zeros
    for b_local in range(2):
        dis_b, reg_b, C_b = per_image[b_local]

        def vrow(t, a):
            return a + vread[pl.ds((b_local * NTILE + t) * 16, 16)]

        vsum = lax.fori_loop(0, NTILE, vrow, zeros)
        var_sum = jnp.full((16,), jnp.sum(vsum), jnp.float32)
        var_b = jnp.where(C_b > 0.0, var_sum / jnp.maximum(C_b, 1.0), 0.0)
        loss = loss + ALPHA * var_b + BETA * dis_b + GAMMA * reg_b

    obuf[...] = jnp.where(iota == 0, loss, 0.0)

    @pl.when(s == 0)
    def _():
        pltpu.sync_copy(obuf, out_hbm.at[c])


def _make_call():
    mesh = plsc.VectorSubcoreMesh(core_axis_name="c", subcore_axis_name="s")
    return pl.kernel(
        _dl_body,
        out_type=jax.ShapeDtypeStruct((2, 16), jnp.float32),
        mesh=mesh,
        compiler_params=pltpu.CompilerParams(needs_layout_passes=False),
        scratch_types=[
            pltpu.VMEM_SHARED((SH_WORDS,), jnp.float32),
            pltpu.VMEM((SDIM, NCLS), jnp.float32),
            pltpu.VMEM((2, FDIM, CROWS, WDIM), jnp.float32),
            pltpu.VMEM((2, CROWS, WDIM), jnp.int32),
            pltpu.VMEM((16,), jnp.float32),
            pltpu.VMEM((2 * NTILE * 16,), jnp.float32),
            pltpu.VMEM((16,), jnp.float32),
            pltpu.SemaphoreType.DMA,
            pltpu.SemaphoreType.DMA,
            pltpu.SemaphoreType.DMA,
            pltpu.SemaphoreType.DMA,
        ],
    )


@jax.jit
def kernel(embeds, labels):
    B, F_, H, W = embeds.shape
    lab = labels.reshape(B, H, W)
    stats = _stats_call(embeds, lab)
    out = _make_call()(embeds, lab, stats)
    return jnp.sum(out)


# speedup vs baseline: 1.2269x; 1.2269x over previous
import functools

import jax
import jax.numpy as jnp
from jax import lax
from jax.experimental import pallas as pl
from jax.experimental.pallas import tpu as pltpu
from jax.experimental.pallas import tpu_sc as plsc

DELTA_V = 0.5
DELTA_D = 1.5
ALPHA = 1.0
BETA = 1.0
GAMMA = 0.001
NCLS = 16
EPS = 1e-12

FDIM = 32
HDIM = 512
WDIM = 512
NTILE = 16
ROWS_PER_TILE = HDIM // NTILE
CROWS = 2
NCHUNK = ROWS_PER_TILE // CROWS
NGRP_ROW = WDIM // 16

RBLK = 16
SDIM = FDIM + 1

VAR_OFF = 0
SH_WORDS = 2 * NTILE * 16


def _stats_body(emb_ref, lab_ref, out_ref):
    rb = pl.program_id(1)
    cls_sub = lax.broadcasted_iota(jnp.int32, (NCLS, WDIM), 0)
    ones_row = jnp.ones((1, WDIM), jnp.float32)
    acc = jnp.zeros((FDIM, NCLS), jnp.float32)
    cnt = jnp.zeros((1, NCLS), jnp.float32)
    for r in range(RBLK):
        lab_r = lab_ref[0, r, :]
        oh16 = (lab_r[None, :] == cls_sub).astype(jnp.float32)
        onehot = jnp.transpose(oh16)
        emb_r = emb_ref[0, :, r, :]
        acc = acc + lax.dot_general(
            emb_r, onehot, (((1,), (0,)), ((), ())),
            preferred_element_type=jnp.float32)
        cnt = cnt + lax.dot_general(
            ones_row, onehot, (((1,), (0,)), ((), ())),
            preferred_element_type=jnp.float32)

    blk = jnp.concatenate([acc, cnt], axis=0)

    @pl.when(rb == 0)
    def _():
        out_ref[0] = jnp.zeros_like(out_ref[0])

    out_ref[0] += blk


def _stats_call(embeds, labels):
    grid = (4, HDIM // RBLK)
    return pl.pallas_call(
        _stats_body,
        grid=grid,
        in_specs=[
            pl.BlockSpec((1, FDIM, RBLK, WDIM), lambda b, r: (b, 0, r, 0)),
            pl.BlockSpec((1, RBLK, WDIM), lambda b, r: (b, r, 0)),
        ],
        out_specs=pl.BlockSpec((1, SDIM, NCLS), lambda b, r: (b, 0, 0)),
        out_shape=jax.ShapeDtypeStruct((4, SDIM, NCLS), jnp.float32),
        compiler_params=pltpu.CompilerParams(
            dimension_semantics=("arbitrary", "arbitrary")),
    )(embeds, labels)


def _take16(vec, idx):
    dn = lax.GatherDimensionNumbers(offset_dims=(), collapsed_slice_dims=(0,),
                                    start_index_map=(0,))
    return lax.gather(vec, idx[:, None], dn, (1,),
                      mode=lax.GatherScatterMode.PROMISE_IN_BOUNDS)


def _sqrt16(x):
    i = plsc.bitcast(x, jnp.int32)
    i = jnp.int32(0x5F3759DF) - (i >> 1)
    r = plsc.bitcast(i, jnp.float32)
    for _ in range(3):
        r = r * (1.5 - 0.5 * x * r * r)
    return x * r


def _dl_body(emb_hbm, lab_hbm, stats_hbm, out_hbm, sh, stat_v, emb_buf,
             lab_buf, varbuf, vread, obuf, sem_e0, sem_e1, sem_l0, sem_l1):
    c = lax.axis_index("c")
    s = lax.axis_index("s")
    sem_e = (sem_e0, sem_e1)
    sem_l = (sem_l0, sem_l1)
    iota = lax.iota(jnp.int32, 16)
    zeros = jnp.zeros((16,), jnp.float32)
    zi = jnp.zeros((16,), jnp.int32)

    def start_at(bb, base, slot):
        pltpu.async_copy(emb_hbm.at[bb, :, pl.ds(base, CROWS), :],
                         emb_buf.at[slot], sem_e[slot])
        pltpu.async_copy(lab_hbm.at[bb, pl.ds(base, CROWS), :],
                         lab_buf.at[slot], sem_l[slot])

    def wait_slot(slot):
        pltpu.make_async_copy(emb_hbm.at[0, :, pl.ds(0, CROWS), :],
                              emb_buf.at[slot], sem_e[slot]).wait()
        pltpu.make_async_copy(lab_hbm.at[0, pl.ds(0, CROWS), :],
                              lab_buf.at[slot], sem_l[slot]).wait()

    def stream_chunks(b, row0, process, carry_init, primed, tail_b,
                      tail_row0):
        if not primed:
            start_at(b, row0, 0)

        def body(kk, car):
            k0 = kk * 2
            start_at(b, row0 + (k0 + 1) * CROWS, 1)
            wait_slot(0)
            car = process(0, car)
            nk = k0 + 2
            last = nk >= NCHUNK
            bb = jnp.where(last, tail_b, b)
            base = jnp.where(last, tail_row0, row0 + nk * CROWS)
            start_at(bb, base, 0)
            wait_slot(1)
            car = process(1, car)
            return car

        return lax.fori_loop(0, NCHUNK // 2, body, carry_init)

    per_image = []
    row0 = s * ROWS_PER_TILE
    start_at(c * 2, row0, 0)
    for b_local in range(2):
        b = c * 2 + b_local

        pltpu.sync_copy(stats_hbm.at[b], stat_v)

        counts = stat_v[FDIM, :]
        present = counts > 0.0
        pres_f = jnp.where(present, 1.0, 0.0).astype(jnp.float32)
        safe = jnp.where(present, counts, 1.0)
        inv_vec = 1.0 / safe
        C_b = jnp.full((16,), jnp.sum(pres_f), jnp.float32)
        mrows = [stat_v[f, :] / safe for f in range(FDIM)]

        nrm2 = zeros
        for f in range(FDIM):
            nrm2 = nrm2 + mrows[f] * mrows[f]
        norms = _sqrt16(nrm2 + EPS)
        reg_sum = jnp.full((16,), jnp.sum(jnp.where(present, norms, 0.0)),
                           jnp.float32)
        reg_b = jnp.where(C_b > 1.0, reg_sum, 0.0)

        def disrow(i, acc):
            d2 = zeros
            for f in range(FDIM):
                mi = _take16(mrows[f], zi + i)
                d = mrows[f] - mi
                d2 = d2 + d * d
            dmat = _sqrt16(d2 + EPS)
            h = jnp.maximum(DELTA_D - dmat, 0.0)
            h = h * h
            pi = _take16(pres_f, zi + i)
            msk = jnp.where(iota > i, pres_f, 0.0) * pi
            return acc + h * msk

        pair_vec = lax.fori_loop(0, NCLS, disrow, zeros)
        pair_sum = jnp.full((16,), jnp.sum(pair_vec), jnp.float32)
        denom = jnp.maximum(C_b * (C_b - 1.0), 1.0)
        dis_b = jnp.where(C_b > 2.0, pair_sum / denom, 0.0)
        per_image.append((dis_b, reg_b, C_b))

        def proc2(slot, vacc):
            def grp(g, va):
                r = g >> 5
                cc = (g & (NGRP_ROW - 1)) * 16
                lab16 = lab_buf[slot, r, pl.ds(cc, 16)]
                accs = [jnp.full((16,), EPS, jnp.float32),
                        zeros, zeros, zeros]
                for f in range(FDIM):
                    v = emb_buf[slot, f, r, pl.ds(cc, 16)]
                    m = _take16(mrows[f], lab16)
                    d = v - m
                    accs[f % 4] = accs[f % 4] + d * d
                acc = (accs[0] + accs[1]) + (accs[2] + accs[3])
                dist = _sqrt16(acc)
                h = jnp.maximum(dist - DELTA_V, 0.0)
                ic = _take16(inv_vec, lab16)
                return va + h * h * ic

            return plsc.parallel_loop(0, CROWS * NGRP_ROW, unroll=2,
                                      carry=vacc)(grp)

        var_vec = stream_chunks(b, row0, proc2, zeros, primed=True,
                                tail_b=jnp.minimum(b + 1, 2 * c + 1),
                                tail_row0=row0)
        varbuf[...] = var_vec
        slot = VAR_OFF + (b_local * NTILE + s) * 16
        pltpu.sync_copy(varbuf, sh.at[pl.ds(slot, 16)])

    wait_slot(0)
    plsc.subcore_barrier()

    pltpu.sync_copy(sh.at[pl.ds(VAR_OFF, 2 * NTILE * 16)], vread)
    loss = zeros
    for b_local in range(2):
        dis_b, reg_b, C_b = per_image[b_local]

        def vrow(t, a):
            return a + vread[pl.ds((b_local * NTILE + t) * 16, 16)]

        vsum = lax.fori_loop(0, NTILE, vrow, zeros)
        var_sum = jnp.full((16,), jnp.sum(vsum), jnp.float32)
        var_b = jnp.where(C_b > 0.0, var_sum / jnp.maximum(C_b, 1.0), 0.0)
        loss = loss + ALPHA * var_b + BETA * dis_b + GAMMA * reg_b

    obuf[...] = jnp.where(iota == 0, loss, 0.0)

    @pl.when(s == 0)
    def _():
        pltpu.sync_copy(obuf, out_hbm.at[c])


def _make_call():
    mesh = plsc.VectorSubcoreMesh(core_axis_name="c", subcore_axis_name="s")
    return pl.kernel(
        _dl_body,
        out_type=jax.ShapeDtypeStruct((2, 16), jnp.float32),
        mesh=mesh,
        compiler_params=pltpu.CompilerParams(needs_layout_passes=False),
        scratch_types=[
            pltpu.VMEM_SHARED((SH_WORDS,), jnp.float32),
            pltpu.VMEM((SDIM, NCLS), jnp.float32),
            pltpu.VMEM((2, FDIM, CROWS, WDIM), jnp.float32),
            pltpu.VMEM((2, CROWS, WDIM), jnp.int32),
            pltpu.VMEM((16,), jnp.float32),
            pltpu.VMEM((2 * NTILE * 16,), jnp.float32),
            pltpu.VMEM((16,), jnp.float32),
            pltpu.SemaphoreType.DMA,
            pltpu.SemaphoreType.DMA,
            pltpu.SemaphoreType.DMA,
            pltpu.SemaphoreType.DMA,
        ],
    )


@jax.jit
def kernel(embeds, labels):
    B, F_, H, W = embeds.shape
    lab = labels.reshape(B, H, W)
    stats = _stats_call(embeds, lab)
    out = _make_call()(embeds, lab, stats)
    return jnp.sum(out)


# speedup vs baseline: 1.4030x; 1.1435x over previous
import functools

import jax
import jax.numpy as jnp
from jax import lax
from jax.experimental import pallas as pl
from jax.experimental.pallas import tpu as pltpu
from jax.experimental.pallas import tpu_sc as plsc

DELTA_V = 0.5
DELTA_D = 1.5
ALPHA = 1.0
BETA = 1.0
GAMMA = 0.001
NCLS = 16
EPS = 1e-12

FDIM = 32
HDIM = 512
WDIM = 512
NTILE = 16
ROWS_PER_TILE = HDIM // NTILE
CROWS = 2
NCHUNK = ROWS_PER_TILE // CROWS
NGRP_ROW = WDIM // 16

RBLK = 32
SDIM = FDIM + 1

VAR_OFF = 0
SH_WORDS = 2 * NTILE * 16


def _stats_body(emb_ref, lab_ref, out_ref):
    rb = pl.program_id(1)
    cls_sub = lax.broadcasted_iota(jnp.int32, (NCLS, WDIM), 0)
    ones_row = jnp.ones((1, WDIM), jnp.float32)
    acc = jnp.zeros((FDIM, NCLS), jnp.float32)
    cnt = jnp.zeros((1, NCLS), jnp.float32)
    for r in range(RBLK):
        lab_r = lab_ref[0, r, :]
        oh16 = (lab_r[None, :] == cls_sub).astype(jnp.float32)
        onehot = jnp.transpose(oh16)
        emb_r = emb_ref[0, :, r, :]
        acc = acc + lax.dot_general(
            emb_r, onehot, (((1,), (0,)), ((), ())),
            preferred_element_type=jnp.float32)
        cnt = cnt + lax.dot_general(
            ones_row, onehot, (((1,), (0,)), ((), ())),
            preferred_element_type=jnp.float32)

    blk = jnp.concatenate([acc, cnt], axis=0)

    @pl.when(rb == 0)
    def _():
        out_ref[0] = jnp.zeros_like(out_ref[0])

    out_ref[0] += blk


def _stats_call(embeds, labels):
    grid = (4, HDIM // RBLK)
    return pl.pallas_call(
        _stats_body,
        grid=grid,
        in_specs=[
            pl.BlockSpec((1, FDIM, RBLK, WDIM), lambda b, r: (b, 0, r, 0)),
            pl.BlockSpec((1, RBLK, WDIM), lambda b, r: (b, r, 0)),
        ],
        out_specs=pl.BlockSpec((1, SDIM, NCLS), lambda b, r: (b, 0, 0)),
        out_shape=jax.ShapeDtypeStruct((4, SDIM, NCLS), jnp.float32),
        compiler_params=pltpu.CompilerParams(
            dimension_semantics=("arbitrary", "arbitrary")),
    )(embeds, labels)


def _take16(vec, idx):
    dn = lax.GatherDimensionNumbers(offset_dims=(), collapsed_slice_dims=(0,),
                                    start_index_map=(0,))
    return lax.gather(vec, idx[:, None], dn, (1,),
                      mode=lax.GatherScatterMode.PROMISE_IN_BOUNDS)


def _sqrt16(x):
    i = plsc.bitcast(x, jnp.int32)
    i = jnp.int32(0x5F3759DF) - (i >> 1)
    r = plsc.bitcast(i, jnp.float32)
    for _ in range(3):
        r = r * (1.5 - 0.5 * x * r * r)
    return x * r


def _dl_body(emb_hbm, lab_hbm, stats_hbm, out_hbm, sh, stat_v, emb_buf,
             lab_buf, varbuf, vread, obuf, sem_e0, sem_e1, sem_l0, sem_l1):
    c = lax.axis_index("c")
    s = lax.axis_index("s")
    sem_e = (sem_e0, sem_e1)
    sem_l = (sem_l0, sem_l1)
    iota = lax.iota(jnp.int32, 16)
    zeros = jnp.zeros((16,), jnp.float32)
    zi = jnp.zeros((16,), jnp.int32)

    def start_at(bb, base, slot):
        pltpu.async_copy(emb_hbm.at[bb, :, pl.ds(base, CROWS), :],
                         emb_buf.at[slot], sem_e[slot])
        pltpu.async_copy(lab_hbm.at[bb, pl.ds(base, CROWS), :],
                         lab_buf.at[slot], sem_l[slot])

    def wait_slot(slot):
        pltpu.make_async_copy(emb_hbm.at[0, :, pl.ds(0, CROWS), :],
                              emb_buf.at[slot], sem_e[slot]).wait()
        pltpu.make_async_copy(lab_hbm.at[0, pl.ds(0, CROWS), :],
                              lab_buf.at[slot], sem_l[slot]).wait()

    def stream_chunks(b, row0, process, carry_init, primed, tail_b,
                      tail_row0):
        if not primed:
            start_at(b, row0, 0)

        def body(kk, car):
            k0 = kk * 2
            start_at(b, row0 + (k0 + 1) * CROWS, 1)
            wait_slot(0)
            car = process(0, car)
            nk = k0 + 2
            last = nk >= NCHUNK
            bb = jnp.where(last, tail_b, b)
            base = jnp.where(last, tail_row0, row0 + nk * CROWS)
            start_at(bb, base, 0)
            wait_slot(1)
            car = process(1, car)
            return car

        return lax.fori_loop(0, NCHUNK // 2, body, carry_init)

    per_image = []
    row0 = s * ROWS_PER_TILE
    start_at(c * 2, row0, 0)
    for b_local in range(2):
        b = c * 2 + b_local

        pltpu.sync_copy(stats_hbm.at[b], stat_v)

        counts = stat_v[FDIM, :]
        present = counts > 0.0
        pres_f = jnp.where(present, 1.0, 0.0).astype(jnp.float32)
        safe = jnp.where(present, counts, 1.0)
        inv_vec = 1.0 / safe
        C_b = jnp.full((16,), jnp.sum(pres_f), jnp.float32)
        mrows = [stat_v[f, :] / safe for f in range(FDIM)]

        nrm2 = zeros
        for f in range(FDIM):
            nrm2 = nrm2 + mrows[f] * mrows[f]
        norms = _sqrt16(nrm2 + EPS)
        reg_sum = jnp.full((16,), jnp.sum(jnp.where(present, norms, 0.0)),
                           jnp.float32)
        reg_b = jnp.where(C_b > 1.0, reg_sum, 0.0)

        def disrow(i, acc):
            d2 = zeros
            for f in range(FDIM):
                mi = _take16(mrows[f], zi + i)
                d = mrows[f] - mi
                d2 = d2 + d * d
            dmat = _sqrt16(d2 + EPS)
            h = jnp.maximum(DELTA_D - dmat, 0.0)
            h = h * h
            pi = _take16(pres_f, zi + i)
            msk = jnp.where(iota > i, pres_f, 0.0) * pi
            return acc + h * msk

        pair_vec = lax.fori_loop(0, NCLS, disrow, zeros)
        pair_sum = jnp.full((16,), jnp.sum(pair_vec), jnp.float32)
        denom = jnp.maximum(C_b * (C_b - 1.0), 1.0)
        dis_b = jnp.where(C_b > 2.0, pair_sum / denom, 0.0)
        per_image.append((dis_b, reg_b, C_b))

        def proc2(slot, vacc):
            def grp(g, va):
                r = g >> 5
                cc = (g & (NGRP_ROW - 1)) * 16
                lab16 = lab_buf[slot, r, pl.ds(cc, 16)]
                accs = [jnp.full((16,), EPS, jnp.float32),
                        zeros, zeros, zeros]
                for f in range(FDIM):
                    v = emb_buf[slot, f, r, pl.ds(cc, 16)]
                    m = _take16(mrows[f], lab16)
                    d = v - m
                    accs[f % 4] = accs[f % 4] + d * d
                acc = (accs[0] + accs[1]) + (accs[2] + accs[3])
                dist = _sqrt16(acc)
                h = jnp.maximum(dist - DELTA_V, 0.0)
                ic = _take16(inv_vec, lab16)
                return va + h * h * ic

            return plsc.parallel_loop(0, CROWS * NGRP_ROW, unroll=2,
                                      carry=vacc)(grp)

        var_vec = stream_chunks(b, row0, proc2, zeros, primed=True,
                                tail_b=jnp.minimum(b + 1, 2 * c + 1),
                                tail_row0=row0)
        varbuf[...] = var_vec
        slot = VAR_OFF + (b_local * NTILE + s) * 16
        pltpu.sync_copy(varbuf, sh.at[pl.ds(slot, 16)])

    wait_slot(0)
    plsc.subcore_barrier()

    pltpu.sync_copy(sh.at[pl.ds(VAR_OFF, 2 * NTILE * 16)], vread)
    loss = zeros
    for b_local in range(2):
        dis_b, reg_b, C_b = per_image[b_local]

        def vrow(t, a):
            return a + vread[pl.ds((b_local * NTILE + t) * 16, 16)]

        vsum = lax.fori_loop(0, NTILE, vrow, zeros)
        var_sum = jnp.full((16,), jnp.sum(vsum), jnp.float32)
        var_b = jnp.where(C_b > 0.0, var_sum / jnp.maximum(C_b, 1.0), 0.0)
        loss = loss + ALPHA * var_b + BETA * dis_b + GAMMA * reg_b

    obuf[...] = jnp.where(iota == 0, loss, 0.0)

    @pl.when(s == 0)
    def _():
        pltpu.sync_copy(obuf, out_hbm.at[c])


def _make_call():
    mesh = plsc.VectorSubcoreMesh(core_axis_name="c", subcore_axis_name="s")
    return pl.kernel(
        _dl_body,
        out_type=jax.ShapeDtypeStruct((2, 16), jnp.float32),
        mesh=mesh,
        compiler_params=pltpu.CompilerParams(needs_layout_passes=False),
        scratch_types=[
            pltpu.VMEM_SHARED((SH_WORDS,), jnp.float32),
            pltpu.VMEM((SDIM, NCLS), jnp.float32),
            pltpu.VMEM((2, FDIM, CROWS, WDIM), jnp.float32),
            pltpu.VMEM((2, CROWS, WDIM), jnp.int32),
            pltpu.VMEM((16,), jnp.float32),
            pltpu.VMEM((2 * NTILE * 16,), jnp.float32),
            pltpu.VMEM((16,), jnp.float32),
            pltpu.SemaphoreType.DMA,
            pltpu.SemaphoreType.DMA,
            pltpu.SemaphoreType.DMA,
            pltpu.SemaphoreType.DMA,
        ],
    )


@jax.jit
def kernel(embeds, labels):
    B, F_, H, W = embeds.shape
    lab = labels.reshape(B, H, W)
    stats = _stats_call(embeds, lab)
    out = _make_call()(embeds, lab, stats)
    return jnp.sum(out)
